# baseline (speedup 1.0000x reference)
import functools

import jax
import jax.numpy as jnp
from jax import lax
from jax.experimental import pallas as pl
from jax.experimental.pallas import tpu as pltpu
from jax.experimental.pallas import tpu_sc as plsc

D_MODEL = 64
SCALE = 8.0
NC, NS = 2, 16
NW = NC * NS
CH = 128
LANES = 16


def kernel(x, table):
    B0, S = x.shape
    B = B0 * S
    assert B % (NW * CH) == 0
    b_per_w = B // NW
    n_ch = b_per_w // CH
    idx = x.reshape(NW, n_ch, CH)
    if idx.dtype != jnp.int32:
        idx = idx.astype(jnp.int32)

    mesh = plsc.VectorSubcoreMesh(core_axis_name="c", subcore_axis_name="s")

    @functools.partial(
        pl.kernel,
        mesh=mesh,
        out_type=jax.ShapeDtypeStruct((B, D_MODEL), jnp.float32),
        scratch_types=[
            pltpu.VMEM((n_ch, CH), jnp.int32),
            pltpu.VMEM((2, CH, D_MODEL), jnp.float32),
            pltpu.SemaphoreType.DMA,
            pltpu.SemaphoreType.DMA,
        ],
        compiler_params=pltpu.CompilerParams(use_tc_tiling_on_sc=False),
    )
    def emb(idx_hbm, table_hbm, out_hbm, idx_v, rows_v, sem0, sem1):
        sems = (sem0, sem1)
        wid = lax.axis_index("s") * NC + lax.axis_index("c")
        base = wid * b_per_w
        pltpu.sync_copy(idx_hbm.at[wid], idx_v)
        for b in range(2):
            pltpu.async_copy(table_hbm.at[idx_v.at[b]], rows_v.at[b], sems[b])

        def pair(i, carry):
            g0 = i * 2
            for b in range(2):
                g = g0 + b
                buf = rows_v.at[b]
                pltpu.make_async_copy(
                    table_hbm.at[idx_v.at[g]], buf, sems[b]).wait()

                def mul_row(r, c2, buf=buf):
                    for c in range(D_MODEL // LANES):
                        sl = (r, pl.ds(c * LANES, LANES))
                        buf[sl] = buf[sl] * SCALE
                    return c2

                lax.fori_loop(0, CH, mul_row, 0, unroll=4)
                pltpu.sync_copy(buf, out_hbm.at[pl.ds(base + g * CH, CH)])

                @pl.when(g + 2 < n_ch)
                def _(b=b, g=g, buf=buf):
                    pltpu.async_copy(
                        table_hbm.at[idx_v.at[g + 2]], buf, sems[b])

            return carry

        lax.fori_loop(0, n_ch // 2, pair, 0)

    out = emb(idx, table)
    return out.reshape(B0, S, D_MODEL)


# speedup vs baseline: 1.0355x; 1.0355x over previous
import functools

import jax
import jax.numpy as jnp
from jax import lax
from jax.experimental import pallas as pl
from jax.experimental.pallas import tpu as pltpu
from jax.experimental.pallas import tpu_sc as plsc

D_MODEL = 64
SCALE = 8.0
NC, NS = 2, 16
NW = NC * NS
CH = 128
K = 4
NBUF = 2
LANES = 16


def kernel(x, table):
    B0, S = x.shape
    B = B0 * S
    assert B % (NW * CH * K) == 0
    b_per_w = B // NW
    n_ch = b_per_w // CH
    n_w = n_ch // K
    idx = x.reshape(NW, n_ch, CH)
    if idx.dtype != jnp.int32:
        idx = idx.astype(jnp.int32)

    mesh = plsc.VectorSubcoreMesh(core_axis_name="c", subcore_axis_name="s")

    @functools.partial(
        pl.kernel,
        mesh=mesh,
        out_type=jax.ShapeDtypeStruct((NW, n_w, K, CH, D_MODEL), jnp.float32),
        scratch_types=[
            pltpu.VMEM((n_ch, CH), jnp.int32),
            pltpu.VMEM((NBUF, K, CH, D_MODEL), jnp.float32),
            pltpu.SemaphoreType.DMA,
            pltpu.SemaphoreType.DMA,
        ],
        compiler_params=pltpu.CompilerParams(use_tc_tiling_on_sc=False),
    )
    def emb(idx_hbm, table_hbm, out_hbm, idx_v, rows_v, sem0, sem1):
        sems = (sem0, sem1)
        wid = lax.axis_index("s") * NC + lax.axis_index("c")
        pltpu.sync_copy(idx_hbm.at[wid], idx_v)

        def fire(w, b):
            for k in range(K):
                pltpu.async_copy(
                    table_hbm.at[idx_v.at[w * K + k]], rows_v.at[b, k],
                    sems[b])

        for b in range(NBUF):
            fire(b, b)

        def wave(i, carry):
            for b in range(NBUF):
                w = i * NBUF + b
                for k in range(K):
                    pltpu.make_async_copy(
                        table_hbm.at[idx_v.at[w * K + k]], rows_v.at[b, k],
                        sems[b]).wait()
                for k in range(K):
                    bufk = rows_v.at[b, k]

                    @plsc.parallel_loop(0, CH, unroll=8)
                    def _(r, bufk=bufk):
                        for c in range(D_MODEL // LANES):
                            sl = (r, pl.ds(c * LANES, LANES))
                            bufk[sl] = bufk[sl] * SCALE

                pltpu.sync_copy(rows_v.at[b], out_hbm.at[wid, w])

                @pl.when(w + NBUF < n_w)
                def _(w=w, b=b):
                    fire(w + NBUF, b)

            return carry

        lax.fori_loop(0, n_w // NBUF, wave, 0)

    out = emb(idx, table)
    return out.reshape(B0, S, D_MODEL)
